# Initial kernel scaffold; baseline (speedup 1.0000x reference)
import jax
import jax.numpy as jnp
from jax.experimental import pallas as pl


def kernel(x, edge_index, relations, concs, W0, b0, Wr, br):
    raise NotImplementedError("write your pallas kernel here")


# speedup vs baseline: 3.0770x; 3.0770x over previous
import functools

import jax
import jax.numpy as jnp
from jax import lax
from jax.experimental import pallas as pl
from jax.experimental.pallas import tpu as pltpu
from jax.experimental.pallas import tpu_sc as plsc

N_NODES = 10000
E = 320000
D = 128
HID = 16
NREL = 5

NC, NS = 2, 16
NW = NC * NS
EPW = E // NW
CH = 80
NCHUNK = EPW // CH

BT = 4000
NBT = E // BT


def _sc_gather_mul(x, row, col):
    mesh = plsc.VectorSubcoreMesh(
        core_axis_name="c", subcore_axis_name="s", num_cores=NC, num_subcores=NS)

    @functools.partial(
        pl.kernel,
        out_type=jax.ShapeDtypeStruct((E, D), jnp.float32),
        mesh=mesh,
        scratch_types=[
            pltpu.VMEM((EPW,), jnp.int32),
            pltpu.VMEM((EPW,), jnp.int32),
            pltpu.VMEM((CH, D), jnp.float32),
            pltpu.VMEM((CH, D), jnp.float32),
            pltpu.SemaphoreType.DMA,
            pltpu.SemaphoreType.DMA,
        ],
    )
    def k(x_hbm, row_hbm, col_hbm, y_hbm, idxr, idxc, xi, xj, sem1, sem2):
        wid = lax.axis_index("s") * NC + lax.axis_index("c")
        base = pl.multiple_of(wid * EPW, EPW)
        pltpu.sync_copy(row_hbm.at[pl.ds(base, EPW)], idxr)
        pltpu.sync_copy(col_hbm.at[pl.ds(base, EPW)], idxc)

        def chunk(i, carry):
            off = pl.multiple_of(i * CH, CH)
            cp1 = pltpu.async_copy(x_hbm.at[idxr.at[pl.ds(off, CH)]], xi, sem1)
            cp2 = pltpu.async_copy(x_hbm.at[idxc.at[pl.ds(off, CH)]], xj, sem2)
            cp1.wait()
            cp2.wait()

            def rowmul(r, c2):
                for kk in range(D // 16):
                    s = pl.ds(kk * 16, 16)
                    xi[r, s] = xi[r, s] * xj[r, s]
                return c2

            lax.fori_loop(0, CH, rowmul, 0, unroll=2)
            pltpu.sync_copy(xi, y_hbm.at[pl.ds(base + off, CH)])
            return carry

        lax.fori_loop(0, NCHUNK, chunk, 0)

    return k(x, row, col)


def _tc_mlp(y, concs, rel2d, W0a, wc, b0r, Wf, bf):
    def body(y_ref, concs_ref, rel_ref, W0a_ref, wc_ref, b0_ref, Wf_ref,
             bf_ref, o_ref):
        yb = y_ref[...]
        cc = concs_ref[...]
        c = cc[:, 0:1] * cc[:, 1:2]
        h = jnp.dot(yb, W0a_ref[...], preferred_element_type=jnp.float32)
        h = h + c * wc_ref[...] + b0_ref[...]
        h = jnp.maximum(h, 0.0)
        o5 = jnp.dot(h, Wf_ref[...], preferred_element_type=jnp.float32)
        o5 = o5 + bf_ref[...]
        rel = rel_ref[...]
        onehot = (rel == lax.broadcasted_iota(jnp.int32, (1, NREL), 1))
        o_ref[...] = jnp.sum(o5 * onehot.astype(jnp.float32), axis=1,
                             keepdims=True)

    return pl.pallas_call(
        body,
        grid=(NBT,),
        in_specs=[
            pl.BlockSpec((BT, D), lambda i: (i, 0)),
            pl.BlockSpec((BT, 2), lambda i: (i, 0)),
            pl.BlockSpec((BT, 1), lambda i: (i, 0)),
            pl.BlockSpec((D, HID), lambda i: (0, 0)),
            pl.BlockSpec((1, HID), lambda i: (0, 0)),
            pl.BlockSpec((1, HID), lambda i: (0, 0)),
            pl.BlockSpec((HID, NREL), lambda i: (0, 0)),
            pl.BlockSpec((1, NREL), lambda i: (0, 0)),
        ],
        out_specs=pl.BlockSpec((BT, 1), lambda i: (i, 0)),
        out_shape=jax.ShapeDtypeStruct((E, 1), jnp.float32),
    )(y, concs, rel2d, W0a, wc, b0r, Wf, bf)


def kernel(x, edge_index, relations, concs, W0, b0, Wr, br):
    row = edge_index[:, 0]
    col = edge_index[:, 1]
    y = _sc_gather_mul(x, row, col)
    W0a = W0[:D]
    wc = W0[D:D + 1, :]
    Wf = Wr[:, :, 0].T
    bf = br[:, 0][None, :]
    return _tc_mlp(y, concs, relations[:, None], W0a, wc, b0[None, :], Wf, bf)


# speedup vs baseline: 4.2331x; 1.3757x over previous
import functools

import jax
import jax.numpy as jnp
from jax import lax
from jax.experimental import pallas as pl
from jax.experimental.pallas import tpu as pltpu
from jax.experimental.pallas import tpu_sc as plsc

N_NODES = 10000
E = 320000
D = 128
HID = 16
NREL = 5

NC, NS = 2, 16
NW = NC * NS
EPW = E // NW
CH = 80
NCHUNK = EPW // CH
NBUF = 5
NJ = NCHUNK // NBUF

BT = 8000
NBT = E // BT


def _sc_gather_mul(x, row, col):
    mesh = plsc.VectorSubcoreMesh(
        core_axis_name="c", subcore_axis_name="s", num_cores=NC, num_subcores=NS)

    @functools.partial(
        pl.kernel,
        out_type=jax.ShapeDtypeStruct((E, D), jnp.float32),
        mesh=mesh,
        scratch_types=[
            pltpu.VMEM((EPW,), jnp.int32),
            pltpu.VMEM((EPW,), jnp.int32),
            [pltpu.VMEM((CH, D), jnp.float32) for _ in range(NBUF)],
            [pltpu.VMEM((CH, D), jnp.float32) for _ in range(NBUF)],
            [pltpu.SemaphoreType.DMA for _ in range(NBUF)],
            [pltpu.SemaphoreType.DMA for _ in range(NBUF)],
            [pltpu.SemaphoreType.DMA for _ in range(NBUF)],
        ],
    )
    def k(x_hbm, row_hbm, col_hbm, y_hbm, idxr, idxc, xi, xj, smr, smc, sst):
        wid = lax.axis_index("s") * NC + lax.axis_index("c")
        base = pl.multiple_of(wid * EPW, EPW)
        pltpu.sync_copy(row_hbm.at[pl.ds(base, EPW)], idxr)
        pltpu.sync_copy(col_hbm.at[pl.ds(base, EPW)], idxc)

        def fire(ci, b):
            off = pl.multiple_of(ci * CH, CH)
            pltpu.async_copy(x_hbm.at[idxr.at[pl.ds(off, CH)]], xi[b], smr[b])
            pltpu.async_copy(x_hbm.at[idxc.at[pl.ds(off, CH)]], xj[b], smc[b])

        def gwait(b):
            pltpu.make_async_copy(x_hbm.at[idxr.at[pl.ds(0, CH)]], xi[b],
                                  smr[b]).wait()
            pltpu.make_async_copy(x_hbm.at[idxc.at[pl.ds(0, CH)]], xj[b],
                                  smc[b]).wait()

        def swait(b):
            pltpu.make_async_copy(xi[b], y_hbm.at[pl.ds(base, CH)],
                                  sst[b]).wait()

        for b in range(NBUF - 1):
            fire(b, b)

        def outer(j, carry):
            for b in range(NBUF):
                ci = j * NBUF + b
                gwait(b)

                def rowmul(r, c2):
                    for kk in range(D // 16):
                        s = pl.ds(kk * 16, 16)
                        xi[b][r, s] = xi[b][r, s] * xj[b][r, s]
                    return c2

                lax.fori_loop(0, CH, rowmul, 0, unroll=2)
                off = pl.multiple_of(ci * CH, CH)
                pltpu.async_copy(xi[b], y_hbm.at[pl.ds(base + off, CH)],
                                 sst[b])
                nb = (b + NBUF - 1) % NBUF
                if b == 0:
                    @pl.when(j > 0)
                    def _():
                        swait(nb)
                    fire(ci + NBUF - 1, nb)
                else:
                    @pl.when(j < NJ - 1)
                    def _():
                        swait(nb)
                        fire(ci + NBUF - 1, nb)
            return carry

        lax.fori_loop(0, NJ, outer, 0)
        for b in range(NBUF):
            swait(b)

    return k(x, row, col)


def _tc_mlp(y, concs, rel2d, W0a, wc, b0r, Wf, bf):
    def body(y_ref, concs_ref, rel_ref, W0a_ref, wc_ref, b0_ref, Wf_ref,
             bf_ref, ones_ref, o_ref):
        yb = y_ref[...]
        cc = concs_ref[...]
        c = cc[:, 0:1] * cc[:, 1:2]
        h = jnp.dot(yb, W0a_ref[...], preferred_element_type=jnp.float32)
        h = h + c * wc_ref[...] + b0_ref[...]
        h = jnp.maximum(h, 0.0)
        o5 = jnp.dot(h, Wf_ref[...], preferred_element_type=jnp.float32)
        o5 = o5 + bf_ref[...]
        rel = rel_ref[...]
        onehot = (rel == lax.broadcasted_iota(jnp.int32, (1, NREL), 1))
        sel = o5 * onehot.astype(jnp.float32)
        o_ref[...] = jnp.dot(sel, ones_ref[...],
                             preferred_element_type=jnp.float32)

    ones5 = jnp.ones((NREL, 1), jnp.float32)
    return pl.pallas_call(
        body,
        grid=(NBT,),
        in_specs=[
            pl.BlockSpec((BT, D), lambda i: (i, 0)),
            pl.BlockSpec((BT, 2), lambda i: (i, 0)),
            pl.BlockSpec((BT, 1), lambda i: (i, 0)),
            pl.BlockSpec((D, HID), lambda i: (0, 0)),
            pl.BlockSpec((1, HID), lambda i: (0, 0)),
            pl.BlockSpec((1, HID), lambda i: (0, 0)),
            pl.BlockSpec((HID, NREL), lambda i: (0, 0)),
            pl.BlockSpec((1, NREL), lambda i: (0, 0)),
            pl.BlockSpec((NREL, 1), lambda i: (0, 0)),
        ],
        out_specs=pl.BlockSpec((BT, 1), lambda i: (i, 0)),
        out_shape=jax.ShapeDtypeStruct((E, 1), jnp.float32),
    )(y, concs, rel2d, W0a, wc, b0r, Wf, bf, ones5)


def kernel(x, edge_index, relations, concs, W0, b0, Wr, br):
    row = edge_index[:, 0]
    col = edge_index[:, 1]
    y = _sc_gather_mul(x, row, col)
    W0a = W0[:D]
    wc = W0[D:D + 1, :]
    Wf = Wr[:, :, 0].T
    bf = br[:, 0][None, :]
    return _tc_mlp(y, concs, relations[:, None], W0a, wc, b0[None, :], Wf, bf)
